# baseline (speedup 1.0000x reference)
import jax
import jax.numpy as jnp
from jax import lax
from jax.experimental import pallas as pl
from jax.experimental.pallas import tpu as pltpu
from jax.experimental.pallas import tpu_sc as plsc

B = 16
N = 1000
L = 8000
LANES = 16
UNROLL = 4
NCH = 5
CL = L // NCH
NPAD = 1024
BOXPAD = 4096


def _body(pt_hbm, boxes_hbm, scores_hbm, classes_hbm, counts_hbm,
          cols_v, boxes_v, scores_v, classes_v, counts_v, sem, sem2):
    b = lax.axis_index("s")
    if True:
        sems = [sem, sem2]

        def copy_chunk(j):
            return [pltpu.async_copy(
                pt_hbm.at[pl.ds(col * L + j * CL, CL)],
                cols_v.at[pl.ds(col * L + j * CL, CL)],
                sems[j % 2]) for col in range(7)]

        cps = copy_chunk(0)

        zf = jnp.zeros((LANES,), jnp.float32)
        zi = jnp.zeros((LANES,), jnp.int32)

        def zero_boxes(i, carry):
            boxes_v[pl.ds(i * LANES, LANES)] = zf
            return carry

        lax.fori_loop(0, BOXPAD // LANES, zero_boxes, 0)

        def zero_sc(i, carry):
            scores_v[pl.ds(i * LANES, LANES)] = zf
            classes_v[pl.ds(i * LANES, LANES)] = zi
            return carry

        lax.fori_loop(0, NPAD // LANES, zero_sc, 0)

        bf = b.astype(jnp.float32)

        def make_step(j):
            def step(i, off):
                base0 = j * CL + i * (LANES * UNROLL)
                masks, incls = [], []
                for k in range(UNROLL):
                    vb = cols_v[pl.ds(base0 + k * LANES, LANES)]
                    masks.append(vb == bf)
                for k in range(UNROLL):
                    incls.append(
                        jnp.cumsum(jnp.where(masks[k], 1, 0).astype(jnp.int32)))
                for k in range(UNROLL):
                    base = base0 + k * LANES
                    ranks = off + incls[k] - 1
                    m2 = jnp.logical_and(masks[k], ranks < N)
                    idx4 = ranks * 4
                    for col in range(4):
                        x = cols_v[pl.ds((1 + col) * L + base, LANES)]
                        plsc.store_scatter(boxes_v, [idx4 + col], x, mask=m2)
                    xs = cols_v[pl.ds(5 * L + base, LANES)]
                    plsc.store_scatter(scores_v, [ranks], xs, mask=m2)
                    xc = cols_v[pl.ds(6 * L + base, LANES)].astype(jnp.int32)
                    plsc.store_scatter(classes_v, [ranks], xc, mask=m2)
                    off = off + jnp.max(incls[k])
                return off
            return step

        off = jnp.zeros((LANES,), jnp.int32)
        for j in range(NCH):
            nxt = copy_chunk(j + 1) if j + 1 < NCH else []
            for cp in cps:
                cp.wait()
            off = lax.fori_loop(0, CL // (LANES * UNROLL), make_step(j), off)
            cps = nxt
        for j in range(128 // LANES):
            counts_v[pl.ds(j * LANES, LANES)] = off

        pltpu.sync_copy(boxes_v, boxes_hbm.at[b])
        pltpu.sync_copy(scores_v, scores_hbm.at[b])
        pltpu.sync_copy(classes_v, classes_hbm.at[b])
        pltpu.sync_copy(counts_v, counts_hbm.at[b])


def kernel(predictions):
    pt = predictions.T.reshape(-1)
    mesh = plsc.VectorSubcoreMesh(
        core_axis_name="c", subcore_axis_name="s", num_cores=1)
    k = pl.kernel(
        _body,
        mesh=mesh,
        compiler_params=pltpu.CompilerParams(needs_layout_passes=False),
        out_type=[
            jax.ShapeDtypeStruct((B, BOXPAD), jnp.float32),
            jax.ShapeDtypeStruct((B, NPAD), jnp.float32),
            jax.ShapeDtypeStruct((B, NPAD), jnp.int32),
            jax.ShapeDtypeStruct((B, 128), jnp.int32),
        ],
        scratch_types=[
            pltpu.VMEM((7 * L,), jnp.float32),
            pltpu.VMEM((BOXPAD,), jnp.float32),
            pltpu.VMEM((NPAD,), jnp.float32),
            pltpu.VMEM((NPAD,), jnp.int32),
            pltpu.VMEM((128,), jnp.int32),
            pltpu.SemaphoreType.DMA,
            pltpu.SemaphoreType.DMA,
        ],
    )
    boxes, scores, classes, counts = k(pt)
    num_predictions = counts[:, :1]
    pred_boxes = boxes[:, :4 * N].reshape(B, N, 4)
    return (num_predictions, pred_boxes, scores[:, :N], classes[:, :N])


# speedup vs baseline: 1.0563x; 1.0563x over previous
import jax
import jax.numpy as jnp
from jax import lax
from jax.experimental import pallas as pl
from jax.experimental.pallas import tpu as pltpu
from jax.experimental.pallas import tpu_sc as plsc

B = 16
N = 1000
L = 8000
LANES = 16
UNROLL = 4
CHUNKS = L // (LANES * UNROLL)
NPAD = 1024
BOXPAD = 4096


def _body(pt_hbm, boxes_hbm, scores_hbm, classes_hbm, counts_hbm,
          cols_v, boxes_v, scores_v, classes_v, counts_v, sem):
    b = lax.axis_index("s")
    if True:
        cp = pltpu.async_copy(pt_hbm, cols_v, sem)

        zf = jnp.zeros((LANES,), jnp.float32)
        zi = jnp.zeros((LANES,), jnp.int32)

        def zero_boxes(i, carry):
            boxes_v[pl.ds(i * LANES, LANES)] = zf
            return carry

        lax.fori_loop(0, BOXPAD // LANES, zero_boxes, 0)

        def zero_sc(i, carry):
            scores_v[pl.ds(i * LANES, LANES)] = zf
            classes_v[pl.ds(i * LANES, LANES)] = zi
            return carry

        lax.fori_loop(0, NPAD // LANES, zero_sc, 0)
        cp.wait()

        bf = b.astype(jnp.float32)

        def step(i, off):
            base0 = i * (LANES * UNROLL)
            masks, incls = [], []
            for k in range(UNROLL):
                vb = cols_v[pl.ds(base0 + k * LANES, LANES)]
                masks.append(vb == bf)
            for k in range(UNROLL):
                incls.append(
                    jnp.cumsum(jnp.where(masks[k], 1, 0).astype(jnp.int32)))
            for k in range(UNROLL):
                base = base0 + k * LANES
                ranks = off + incls[k] - 1
                m2 = jnp.logical_and(masks[k], ranks < N)
                idx4 = ranks * 4
                for col in range(4):
                    x = cols_v[pl.ds((1 + col) * L + base, LANES)]
                    plsc.store_scatter(boxes_v, [idx4 + col], x, mask=m2)
                xs = cols_v[pl.ds(5 * L + base, LANES)]
                plsc.store_scatter(scores_v, [ranks], xs, mask=m2)
                xc = cols_v[pl.ds(6 * L + base, LANES)].astype(jnp.int32)
                plsc.store_scatter(classes_v, [ranks], xc, mask=m2)
                off = off + jnp.max(incls[k])
            return off

        off = lax.fori_loop(0, CHUNKS, step, jnp.zeros((LANES,), jnp.int32))
        for j in range(128 // LANES):
            counts_v[pl.ds(j * LANES, LANES)] = off

        pltpu.sync_copy(boxes_v, boxes_hbm.at[b])
        pltpu.sync_copy(scores_v, scores_hbm.at[b])
        pltpu.sync_copy(classes_v, classes_hbm.at[b])
        pltpu.sync_copy(counts_v, counts_hbm.at[b])


def kernel(predictions):
    pt = predictions.T.reshape(-1)
    mesh = plsc.VectorSubcoreMesh(
        core_axis_name="c", subcore_axis_name="s", num_cores=1)
    k = pl.kernel(
        _body,
        mesh=mesh,
        compiler_params=pltpu.CompilerParams(needs_layout_passes=False),
        out_type=[
            jax.ShapeDtypeStruct((B, BOXPAD), jnp.float32),
            jax.ShapeDtypeStruct((B, NPAD), jnp.float32),
            jax.ShapeDtypeStruct((B, NPAD), jnp.int32),
            jax.ShapeDtypeStruct((B, 128), jnp.int32),
        ],
        scratch_types=[
            pltpu.VMEM((7 * L,), jnp.float32),
            pltpu.VMEM((BOXPAD,), jnp.float32),
            pltpu.VMEM((NPAD,), jnp.float32),
            pltpu.VMEM((NPAD,), jnp.int32),
            pltpu.VMEM((128,), jnp.int32),
            pltpu.SemaphoreType.DMA,
        ],
    )
    boxes, scores, classes, counts = k(pt)
    num_predictions = counts[:, :1]
    pred_boxes = boxes[:, :4 * N].reshape(B, N, 4)
    return (num_predictions, pred_boxes, scores[:, :N], classes[:, :N])
